# baseline (speedup 1.0000x reference)
import functools

import jax
import jax.numpy as jnp
from jax import lax
from jax.experimental import pallas as pl
from jax.experimental.pallas import tpu as pltpu
from jax.experimental.pallas import tpu_sc as plsc

NUM_STATE = 1000000
EMBED_DIM = 16
BATCH = 16384
FIELDS = 100

NC = 2
NS = 16
NW = NC * NS
BH = BATCH // 128
TPB = BH // NW
ROWS_W = TPB * 128
NT = FIELDS * TPB
EH = EMBED_DIM // 8

assert BATCH % (128 * NW) == 0 and NT % 2 == 0


def _iota16():
  return lax.iota(jnp.int32, 16)


def _gather_body(idx_hbm, table_hbm, out_hbm, slab, idxst, rows, trans,
                 gsem, osem):
  wid = lax.axis_index("s") * NC + lax.axis_index("c")
  b0 = wid * ROWS_W
  pltpu.sync_copy(idx_hbm.at[pl.ds(b0, ROWS_W)], slab)

  def tile_fb(t):
    t = jnp.minimum(t, NT - 1)
    return t // TPB, t % TPB

  def stage_col(t, b):
    f, bl = tile_fb(t)
    fv = jnp.full((16,), 0, jnp.int32) + f
    vals = [
        plsc.load_gather(slab, [bl * 128 + k * 16 + _iota16(), fv])
        for k in range(8)
    ]
    for k in range(8):
      idxst[b, pl.ds(k * 16, 16)] = vals[k]

  def fire_gather(b):
    pltpu.async_copy(table_hbm.at[idxst.at[b]], rows.at[b], gsem.at[b])

  stage_col(0, 0)
  fire_gather(0)

  def pair(g, _):
    for b in range(2):
      t = g * 2 + b
      f, bl = tile_fb(t)
      bh = bl
      pltpu.make_async_copy(
          table_hbm.at[idxst.at[b]], rows.at[b], gsem.at[b]
      ).wait()
      stage_col(t + 1, 1 - b)
      fire_gather(1 - b)
      @pl.when(g >= 1)
      def _():
        for eh in range(EH):
          pltpu.make_async_copy(
              trans.at[b].at[eh], out_hbm.at[0, eh, 0], osem.at[b]
          ).wait()
      r2 = rows.at[b]
      for k in range(8):
        rv = k * 16 + _iota16()
        for e in range(EMBED_DIM):
          c = (e + _iota16()) & 15
          v = plsc.load_gather(r2, [rv, c])
          plsc.store_scatter(trans.at[b], [c >> 3, c & 7, rv], v)
      for eh in range(EH):
        pltpu.async_copy(
            trans.at[b].at[eh],
            out_hbm.at[f, eh, wid * TPB + bh],
            osem.at[b],
        )
    return 0

  lax.fori_loop(0, NT // 2, pair, 0)

  for b in range(2):
    for eh in range(EH):
      pltpu.make_async_copy(
          trans.at[b].at[eh], out_hbm.at[0, eh, 0], osem.at[b]
      ).wait()
  pltpu.make_async_copy(
      table_hbm.at[idxst.at[0]], rows.at[0], gsem.at[0]
  ).wait()


def _gather(idx, table):
  k = functools.partial(
      pl.kernel,
      out_type=jax.ShapeDtypeStruct((FIELDS, EH, BH, 8, 128), jnp.float32),
      mesh=plsc.VectorSubcoreMesh(core_axis_name="c", subcore_axis_name="s"),
      scratch_types=[
          pltpu.VMEM((ROWS_W, FIELDS), jnp.int32),
          pltpu.VMEM((2, 128), jnp.int32),
          pltpu.VMEM((2, 128, EMBED_DIM), jnp.float32),
          pltpu.VMEM((2, EH, 8, 128), jnp.float32),
          pltpu.SemaphoreType.DMA((2,)),
          pltpu.SemaphoreType.DMA((2,)),
      ],
      compiler_params=pltpu.CompilerParams(
          use_tc_tiling_on_sc=False, needs_layout_passes=False
      ),
  )(_gather_body)
  return k(idx, table)


def kernel(inputs, table):
  r5 = _gather(inputs, table)
  return r5.transpose(2, 4, 0, 1, 3).reshape(BATCH, FIELDS, EMBED_DIM)


# speedup vs baseline: 1.3991x; 1.3991x over previous
import functools

import jax
import jax.numpy as jnp
from jax import lax
from jax.experimental import pallas as pl
from jax.experimental.pallas import tpu as pltpu
from jax.experimental.pallas import tpu_sc as plsc

NUM_STATE = 1000000
EMBED_DIM = 16
BATCH = 16384
FIELDS = 100

NC = 2
NS = 16
NW = NC * NS
BH = BATCH // 128
TPB = BH // NW
ROWS_W = TPB * 128
NT = FIELDS * TPB
EH = EMBED_DIM // 8
D = 4

assert BATCH % (128 * NW) == 0 and NT % D == 0


def _iota16():
  return lax.iota(jnp.int32, 16)


def _gather_body(idx_hbm, table_hbm, out_hbm, slab, idxst, rows, trans,
                 gsem, osem):
  wid = lax.axis_index("s") * NC + lax.axis_index("c")
  b0 = wid * ROWS_W
  pltpu.sync_copy(idx_hbm.at[pl.ds(b0, ROWS_W)], slab)

  def tile_fb(t):
    t = jnp.minimum(t, NT - 1)
    return t // TPB, t % TPB

  def stage_col(t, b):
    f, bl = tile_fb(t)
    fv = jnp.full((16,), 0, jnp.int32) + f
    vals = [
        plsc.load_gather(slab, [bl * 128 + k * 16 + _iota16(), fv])
        for k in range(8)
    ]
    for k in range(8):
      idxst[b, pl.ds(k * 16, 16)] = vals[k]

  def fire_gather(b):
    pltpu.async_copy(table_hbm.at[idxst.at[b]], rows.at[b], gsem.at[b])

  for s in range(D - 1):
    stage_col(s, s)
    fire_gather(s)

  def quad(g, _):
    for u in range(D):
      t = g * D + u
      f, bl = tile_fb(t)
      pltpu.make_async_copy(
          table_hbm.at[idxst.at[u]], rows.at[u], gsem.at[u]
      ).wait()
      s = (u + D - 1) % D
      stage_col(t + D - 1, s)
      fire_gather(s)
      @pl.when(g >= 1)
      def _():
        for eh in range(EH):
          pltpu.make_async_copy(
              trans.at[u].at[eh], out_hbm.at[0, eh, 0], osem.at[u]
          ).wait()
      r2 = rows.at[u]
      for k in range(8):
        rv = k * 16 + _iota16()
        vals = [
            plsc.load_gather(r2, [rv, jnp.full((16,), e, jnp.int32)])
            for e in range(EMBED_DIM)
        ]
        for e in range(EMBED_DIM):
          trans[u, e // 8, e % 8, pl.ds(k * 16, 16)] = vals[e]
      for eh in range(EH):
        pltpu.async_copy(
            trans.at[u].at[eh],
            out_hbm.at[f, eh, wid * TPB + bl],
            osem.at[u],
        )
    return 0

  lax.fori_loop(0, NT // D, quad, 0)

  for u in range(D):
    for eh in range(EH):
      pltpu.make_async_copy(
          trans.at[u].at[eh], out_hbm.at[0, eh, 0], osem.at[u]
      ).wait()
  for s in range(D - 1):
    pltpu.make_async_copy(
        table_hbm.at[idxst.at[s]], rows.at[s], gsem.at[s]
    ).wait()


def _gather(idx, table):
  k = functools.partial(
      pl.kernel,
      out_type=jax.ShapeDtypeStruct((FIELDS, EH, BH, 8, 128), jnp.float32),
      mesh=plsc.VectorSubcoreMesh(core_axis_name="c", subcore_axis_name="s"),
      scratch_types=[
          pltpu.VMEM((ROWS_W, FIELDS), jnp.int32),
          pltpu.VMEM((D, 128), jnp.int32),
          pltpu.VMEM((D, 128, EMBED_DIM), jnp.float32),
          pltpu.VMEM((D, EH, 8, 128), jnp.float32),
          pltpu.SemaphoreType.DMA((D,)),
          pltpu.SemaphoreType.DMA((D,)),
      ],
      compiler_params=pltpu.CompilerParams(
          use_tc_tiling_on_sc=False, needs_layout_passes=False
      ),
  )(_gather_body)
  return k(idx, table)


def kernel(inputs, table):
  r5 = _gather(inputs, table)
  return r5.transpose(2, 4, 0, 1, 3).reshape(BATCH, FIELDS, EMBED_DIM)
